# Initial kernel scaffold; baseline (speedup 1.0000x reference)
import jax
import jax.numpy as jnp
from jax.experimental import pallas as pl


def kernel(token_ids, type_ids, token_table, type_table, W, b):
    raise NotImplementedError("write your pallas kernel here")


# speedup vs baseline: 1.2061x; 1.2061x over previous
import functools

import jax
import jax.numpy as jnp
from jax import lax
from jax.experimental import pallas as pl
from jax.experimental.pallas import tpu as pltpu
from jax.experimental.pallas import tpu_sc as plsc

N = 50000
H = 512
VT = 100000
VY = 64

NC = 2
NS = 16
NW = NC * NS

N_PAD = 50176
B_PER_W = N_PAD // NW
K = 112
CHUNKS = B_PER_W // K

TN = 512
NB = N_PAD // TN


def _fused_type_body(tt_ref, wb_ref, b_ref, o_ref):
    o_ref[...] = (
        jnp.dot(tt_ref[...], wb_ref[...],
                preferred_element_type=jnp.float32,
                precision=lax.Precision.HIGHEST)
        + b_ref[...]
    )


def _sc_gather(token_table, ids_pad):
    mesh = plsc.VectorSubcoreMesh(core_axis_name="c", subcore_axis_name="s")

    @functools.partial(
        pl.kernel,
        mesh=mesh,
        out_type=jax.ShapeDtypeStruct((N_PAD, H), jnp.float32),
        scratch_types=[
            pltpu.VMEM((B_PER_W,), jnp.int32),
            pltpu.VMEM((K, H), jnp.float32),
            pltpu.SemaphoreType.DMA,
        ],
    )
    def gather_k(table_hbm, idx_hbm, out_hbm, idx_v, rows_v, sem):
        wid = lax.axis_index("s") * NC + lax.axis_index("c")
        base = wid * B_PER_W
        pltpu.sync_copy(idx_hbm.at[pl.ds(base, B_PER_W)], idx_v)

        def body(c, carry):
            row0 = c * K
            pltpu.async_copy(
                table_hbm.at[idx_v.at[pl.ds(row0, K)]], rows_v, sem
            ).wait()
            pltpu.sync_copy(rows_v, out_hbm.at[pl.ds(base + row0, K)])
            return carry

        lax.fori_loop(0, CHUNKS, body, 0)

    return gather_k(token_table, ids_pad)


def _tc_body(tok_ref, ids_ref, wt_ref, fused_ref, o_ref):
    ids = ids_ref[0, 0, :]
    onehot = (ids[:, None]
              == lax.broadcasted_iota(jnp.int32, (TN, VY), 1)).astype(jnp.float32)
    o_ref[...] = (
        jnp.dot(tok_ref[...], wt_ref[...],
                preferred_element_type=jnp.float32,
                precision=lax.Precision.HIGHEST)
        + jnp.dot(onehot, fused_ref[...],
                  preferred_element_type=jnp.float32,
                  precision=lax.Precision.HIGHEST)
    )


def kernel(token_ids, type_ids, token_table, type_table, W, b):
    w_top = W[:H]
    w_bot = W[H:]

    fused = pl.pallas_call(
        _fused_type_body,
        out_shape=jax.ShapeDtypeStruct((VY, H), jnp.float32),
    )(type_table, w_bot, b.reshape(1, H))

    tok_ids_pad = jnp.pad(token_ids.astype(jnp.int32), (0, N_PAD - N))
    typ_ids_pad = jnp.pad(type_ids.astype(jnp.int32), (0, N_PAD - N))

    tok_rows = _sc_gather(token_table, tok_ids_pad)

    x = pl.pallas_call(
        _tc_body,
        grid=(NB,),
        in_specs=[
            pl.BlockSpec((TN, H), lambda i: (i, 0)),
            pl.BlockSpec((1, 1, TN), lambda i: (i, 0, 0)),
            pl.BlockSpec((H, H), lambda i: (0, 0)),
            pl.BlockSpec((VY, H), lambda i: (0, 0)),
        ],
        out_specs=pl.BlockSpec((TN, H), lambda i: (i, 0)),
        out_shape=jax.ShapeDtypeStruct((N_PAD, H), jnp.float32),
    )(tok_rows, typ_ids_pad.reshape(NB, 1, TN), w_top, fused)

    return x[:N]
